# baseline (speedup 1.0000x reference)
import functools

import jax
import jax.numpy as jnp
from jax import lax
from jax.experimental import pallas as pl
from jax.experimental.pallas import tpu as pltpu
from jax.experimental.pallas import tpu_sc as plsc

B = 2
D = H = W = 128
V = D * H * W
LANES = 16
NC, NS = 2, 16
NW = NC * NS
VPW = B * V // NW
NVOX = 4096
NCHUNK = VPW // NVOX
STEPS = NVOX // LANES


def _tec_body(src_hbm, fz_hbm, fy_hbm, fx_hbm, out_hbm,
              fzv, fyv, fxv,
              zp0, zp1, yp0, yp1, xp0, xp1,
              wz0, wz1, wy0, wy1, wx0, wx1,
              idxv, valv, accv, sem):
    cid = lax.axis_index("c")
    sid = lax.axis_index("s")
    wid = sid * NC + cid
    batch = wid // (NW // B)
    bofs = batch * V
    g0w = wid * VPW
    iota = lax.iota(jnp.int32, LANES)

    def axis_parts(coord_i, frac_part_f, limit, stride_log2, base):
        in0 = (coord_i >= 0) & (coord_i <= limit)
        in1 = (coord_i >= -1) & (coord_i <= limit - 1)
        c0 = jnp.clip(coord_i, 0, limit)
        c1 = jnp.clip(coord_i + 1, 0, limit)
        p0 = base + (c0 << stride_log2)
        p1 = base + (c1 << stride_log2)
        w0 = jnp.where(in0, 1.0 - frac_part_f, 0.0)
        w1 = jnp.where(in1, frac_part_f, 0.0)
        return p0, p1, w0, w1

    def chunk(c, carry):
        g0 = g0w + c * NVOX
        v0 = g0 - bofs
        pltpu.sync_copy(fz_hbm.at[pl.ds(g0, NVOX)], fzv)
        pltpu.sync_copy(fy_hbm.at[pl.ds(g0, NVOX)], fyv)
        pltpu.sync_copy(fx_hbm.at[pl.ds(g0, NVOX)], fxv)

        def pass1(i, carry1):
            sl = pl.ds(i * LANES, LANES)
            v = v0 + i * LANES + iota
            wq = v & (W - 1)
            hq = (v >> 7) & (H - 1)
            dq = v >> 14

            def floor_frac(pos_i, f):
                pos = pos_i.astype(jnp.float32) + f
                t = pos.astype(jnp.int32)
                t = t - jnp.where(t.astype(jnp.float32) > pos, 1, 0)
                return t, pos - t.astype(jnp.float32)

            z0i, frz = floor_frac(dq, fzv[sl])
            y0i, fry = floor_frac(hq, fyv[sl])
            x0i, frx = floor_frac(wq, fxv[sl])

            a, b_, w0, w1 = axis_parts(z0i, frz, D - 1, 14, bofs)
            zp0[sl], zp1[sl], wz0[sl], wz1[sl] = a, b_, w0, w1
            a, b_, w0, w1 = axis_parts(y0i, fry, H - 1, 7, 0)
            yp0[sl], yp1[sl], wy0[sl], wy1[sl] = a, b_, w0, w1
            a, b_, w0, w1 = axis_parts(x0i, frx, W - 1, 0, 0)
            xp0[sl], xp1[sl], wx0[sl], wx1[sl] = a, b_, w0, w1
            return carry1

        lax.fori_loop(0, STEPS, pass1, 0)

        first = True
        for zp, wz in ((zp0, wz0), (zp1, wz1)):
            for yp, wy in ((yp0, wy0), (yp1, wy1)):
                for xp, wx in ((xp0, wx0), (xp1, wx1)):
                    def mkidx(i, carry1, zp=zp, yp=yp, xp=xp):
                        sl = pl.ds(i * LANES, LANES)
                        idxv[sl] = zp[sl] + yp[sl] + xp[sl]
                        return carry1

                    lax.fori_loop(0, STEPS, mkidx, 0)
                    pltpu.async_copy(src_hbm.at[idxv], valv, sem).wait()

                    def accum(i, carry1, wz=wz, wy=wy, wx=wx, first=first):
                        sl = pl.ds(i * LANES, LANES)
                        contrib = (wz[sl] * wy[sl]) * (wx[sl] * valv[sl])
                        if first:
                            accv[sl] = contrib
                        else:
                            accv[sl] = accv[sl] + contrib
                        return carry1

                    lax.fori_loop(0, STEPS, accum, 0)
                    first = False

        pltpu.sync_copy(accv, out_hbm.at[pl.ds(g0, NVOX)])
        return carry

    lax.fori_loop(0, NCHUNK, chunk, 0)


@jax.jit
def kernel(src, flow):
    src_flat = src.reshape(B * V)
    fz = flow[:, 0].reshape(B * V)
    fy = flow[:, 1].reshape(B * V)
    fx = flow[:, 2].reshape(B * V)

    mesh = plsc.VectorSubcoreMesh(core_axis_name="c", subcore_axis_name="s")
    f32 = jnp.float32
    i32 = jnp.int32
    call = functools.partial(
        pl.kernel,
        out_type=jax.ShapeDtypeStruct((B * V,), f32),
        mesh=mesh,
        scratch_types=[
            pltpu.VMEM((NVOX,), f32),
            pltpu.VMEM((NVOX,), f32),
            pltpu.VMEM((NVOX,), f32),
            pltpu.VMEM((NVOX,), i32),
            pltpu.VMEM((NVOX,), i32),
            pltpu.VMEM((NVOX,), i32),
            pltpu.VMEM((NVOX,), i32),
            pltpu.VMEM((NVOX,), i32),
            pltpu.VMEM((NVOX,), i32),
            pltpu.VMEM((NVOX,), f32),
            pltpu.VMEM((NVOX,), f32),
            pltpu.VMEM((NVOX,), f32),
            pltpu.VMEM((NVOX,), f32),
            pltpu.VMEM((NVOX,), f32),
            pltpu.VMEM((NVOX,), f32),
            pltpu.VMEM((NVOX,), i32),
            pltpu.VMEM((NVOX,), f32),
            pltpu.VMEM((NVOX,), f32),
            pltpu.SemaphoreType.DMA,
        ],
    )(_tec_body)
    out = call(src_flat, fz, fy, fx)
    return out.reshape(B, 1, D, H, W)


# speedup vs baseline: 5.4232x; 5.4232x over previous
import functools

import jax
import jax.numpy as jnp
from jax import lax
from jax.experimental import pallas as pl
from jax.experimental.pallas import tpu as pltpu
from jax.experimental.pallas import tpu_sc as plsc

B = 2
D = H = W = 128
V = D * H * W
LANES = 16
NC, NS = 2, 16
NW = NC * NS
DPW = D // (NW // B)

DBLK, HBLK = 4, 16
NVOX = DBLK * HBLK * W
STEPS = NVOX // LANES
HALO = 6
ZWIN = DBLK + 2 * HALO + 2
YWIN = HBLK + 2 * HALO + 2
SLAB = ZWIN * YWIN * W
NSUB = 1024
SSTEPS = NSUB // LANES

f32 = jnp.float32
i32 = jnp.int32


def _floor_frac(pos_i, f):
    pos = pos_i.astype(f32) + f
    t = pos.astype(i32)
    t = t - jnp.where(t.astype(f32) > pos, 1, 0)
    return t, pos - t.astype(f32)


def _tec_body(src_hbm, fz_hbm, fy_hbm, fx_hbm, out_hbm,
              slab, fzv, fyv, fxv, accv,
              zq0, zq1, yq0, yq1, xq0, xq1,
              sz0, sz1, sy0, sy1, sx0, sx1,
              idx2, val2, sem_in, sem_g):
    cid = lax.axis_index("c")
    sid = lax.axis_index("s")
    wid = sid * NC + cid
    batch = wid // (NW // B)
    bofs = batch * V
    d_base = (wid % (NW // B)) * DPW
    iota = lax.iota(i32, LANES)

    def chunk(c, carry):
        d0 = d_base + (c // (H // HBLK)) * DBLK
        h0 = (c % (H // HBLK)) * HBLK
        z_base = jnp.clip(d0 - HALO, 0, D - ZWIN)
        y_base = jnp.clip(h0 - HALO, 0, H - YWIN)

        copies = []
        for zz in range(ZWIN):
            off = bofs + (z_base + zz) * (H * W) + y_base * W
            copies.append(pltpu.async_copy(
                src_hbm.at[pl.ds(off, YWIN * W)],
                slab.at[pl.ds(zz * YWIN * W, YWIN * W)], sem_in))
        for dd in range(DBLK):
            off = bofs + (d0 + dd) * (H * W) + h0 * W
            t = pl.ds(dd * HBLK * W, HBLK * W)
            copies.append(pltpu.async_copy(
                fz_hbm.at[pl.ds(off, HBLK * W)], fzv.at[t], sem_in))
            copies.append(pltpu.async_copy(
                fy_hbm.at[pl.ds(off, HBLK * W)], fyv.at[t], sem_in))
            copies.append(pltpu.async_copy(
                fx_hbm.at[pl.ds(off, HBLK * W)], fxv.at[t], sem_in))
        for cp in copies:
            cp.wait()

        def fast(i, flag):
            sl = pl.ds(i * LANES, LANES)
            j = i * LANES + iota
            wq = j & (W - 1)
            rr = j >> 7
            hq = h0 + (rr & (HBLK - 1))
            dq = d0 + (rr >> 4)

            z0, frz = _floor_frac(dq, fzv[sl])
            y0, fry = _floor_frac(hq, fyv[sl])
            x0, frx = _floor_frac(wq, fxv[sl])

            uz = z0 - z_base
            inz0 = (z0 >= 0) & (z0 <= D - 1)
            inz1 = (z0 >= -1) & (z0 <= D - 2)
            zp0 = jnp.clip(uz, 0, ZWIN - 1) * (YWIN * W)
            zp1 = jnp.clip(uz + 1, 0, ZWIN - 1) * (YWIN * W)
            wz0 = jnp.where(inz0, 1.0 - frz, 0.0)
            wz1 = jnp.where(inz1, frz, 0.0)
            oz = (inz0 & ((uz < 0) | (uz > ZWIN - 1))) | \
                 (inz1 & ((uz < -1) | (uz > ZWIN - 2)))
            uy = y0 - y_base
            iny0 = (y0 >= 0) & (y0 <= H - 1)
            iny1 = (y0 >= -1) & (y0 <= H - 2)
            yp0 = jnp.clip(uy, 0, YWIN - 1) << 7
            yp1 = jnp.clip(uy + 1, 0, YWIN - 1) << 7
            wy0 = jnp.where(iny0, 1.0 - fry, 0.0)
            wy1 = jnp.where(iny1, fry, 0.0)
            oy = (iny0 & ((uy < 0) | (uy > YWIN - 1))) | \
                 (iny1 & ((uy < -1) | (uy > YWIN - 2)))
            inx0 = (x0 >= 0) & (x0 <= W - 1)
            inx1 = (x0 >= -1) & (x0 <= W - 2)
            xp0 = jnp.clip(x0, 0, W - 1)
            xp1 = jnp.clip(x0 + 1, 0, W - 1)
            wx0 = jnp.where(inx0, 1.0 - frx, 0.0)
            wx1 = jnp.where(inx1, frx, 0.0)

            acc = None
            for zp, wz in ((zp0, wz0), (zp1, wz1)):
                for yp, wy in ((yp0, wy0), (yp1, wy1)):
                    bzy = zp + yp
                    tzy = wz * wy
                    for xp, wx in ((xp0, wx0), (xp1, wx1)):
                        val = plsc.load_gather(slab, [bzy + xp])
                        term = (tzy * wx) * val
                        acc = term if acc is None else acc + term
            accv[sl] = acc
            return flag | jnp.where(oz | oy, 1, 0)

        flag = lax.fori_loop(0, STEPS, fast, jnp.zeros((LANES,), i32))

        @pl.when(jnp.max(flag) > 0)
        def _slow():
            def sub_body(sub, carry2):
                def pass1(i, carry1):
                    sl = pl.ds(i * LANES, LANES)
                    j = i * LANES + iota
                    wq = j & (W - 1)
                    rr = sub * (NSUB // W) + (j >> 7)
                    hq = h0 + (rr & (HBLK - 1))
                    dq = d0 + (rr >> 4)
                    z0, frz = _floor_frac(dq, fzv[pl.ds(sub * NSUB + i * LANES, LANES)])
                    y0, fry = _floor_frac(hq, fyv[pl.ds(sub * NSUB + i * LANES, LANES)])
                    x0, frx = _floor_frac(wq, fxv[pl.ds(sub * NSUB + i * LANES, LANES)])

                    def parts(c0i, frac, limit, shift, base, r0, r1, w0r, w1r):
                        in0 = (c0i >= 0) & (c0i <= limit)
                        in1 = (c0i >= -1) & (c0i <= limit - 1)
                        r0[sl] = base + (jnp.clip(c0i, 0, limit) << shift)
                        r1[sl] = base + (jnp.clip(c0i + 1, 0, limit) << shift)
                        w0r[sl] = jnp.where(in0, 1.0 - frac, 0.0)
                        w1r[sl] = jnp.where(in1, frac, 0.0)

                    parts(z0, frz, D - 1, 14, bofs, zq0, zq1, sz0, sz1)
                    parts(y0, fry, H - 1, 7, 0, yq0, yq1, sy0, sy1)
                    parts(x0, frx, W - 1, 0, 0, xq0, xq1, sx0, sx1)
                    return carry1

                lax.fori_loop(0, SSTEPS, pass1, 0)

                first = True
                for zq, sz in ((zq0, sz0), (zq1, sz1)):
                    for yq, sy in ((yq0, sy0), (yq1, sy1)):
                        for xq, sx in ((xq0, sx0), (xq1, sx1)):
                            def mkidx(i, carry1, zq=zq, yq=yq, xq=xq):
                                sl = pl.ds(i * LANES, LANES)
                                idx2[sl] = zq[sl] + yq[sl] + xq[sl]
                                return carry1

                            lax.fori_loop(0, SSTEPS, mkidx, 0)
                            pltpu.async_copy(src_hbm.at[idx2], val2, sem_g).wait()

                            def accum(i, carry1, sz=sz, sy=sy, sx=sx, first=first):
                                sl = pl.ds(i * LANES, LANES)
                                osl = pl.ds(sub * NSUB + i * LANES, LANES)
                                contrib = (sz[sl] * sy[sl]) * (sx[sl] * val2[sl])
                                if first:
                                    accv[osl] = contrib
                                else:
                                    accv[osl] = accv[osl] + contrib
                                return carry1

                            lax.fori_loop(0, SSTEPS, accum, 0)
                            first = False
                return carry2

            lax.fori_loop(0, NVOX // NSUB, sub_body, 0)

        outs = []
        for dd in range(DBLK):
            off = bofs + (d0 + dd) * (H * W) + h0 * W
            outs.append(pltpu.async_copy(
                accv.at[pl.ds(dd * HBLK * W, HBLK * W)],
                out_hbm.at[pl.ds(off, HBLK * W)], sem_in))
        for cp in outs:
            cp.wait()
        return carry

    lax.fori_loop(0, (DPW // DBLK) * (H // HBLK), chunk, 0)


@jax.jit
def kernel(src, flow):
    src_flat = src.reshape(B * V)
    fz = flow[:, 0].reshape(B * V)
    fy = flow[:, 1].reshape(B * V)
    fx = flow[:, 2].reshape(B * V)

    mesh = plsc.VectorSubcoreMesh(core_axis_name="c", subcore_axis_name="s")
    call = functools.partial(
        pl.kernel,
        out_type=jax.ShapeDtypeStruct((B * V,), f32),
        mesh=mesh,
        compiler_params=pltpu.CompilerParams(needs_layout_passes=False),
        scratch_types=[
            pltpu.VMEM((SLAB,), f32),
            pltpu.VMEM((NVOX,), f32),
            pltpu.VMEM((NVOX,), f32),
            pltpu.VMEM((NVOX,), f32),
            pltpu.VMEM((NVOX,), f32),
            pltpu.VMEM((NSUB,), i32),
            pltpu.VMEM((NSUB,), i32),
            pltpu.VMEM((NSUB,), i32),
            pltpu.VMEM((NSUB,), i32),
            pltpu.VMEM((NSUB,), i32),
            pltpu.VMEM((NSUB,), i32),
            pltpu.VMEM((NSUB,), f32),
            pltpu.VMEM((NSUB,), f32),
            pltpu.VMEM((NSUB,), f32),
            pltpu.VMEM((NSUB,), f32),
            pltpu.VMEM((NSUB,), f32),
            pltpu.VMEM((NSUB,), f32),
            pltpu.VMEM((NSUB,), i32),
            pltpu.VMEM((NSUB,), f32),
            pltpu.SemaphoreType.DMA,
            pltpu.SemaphoreType.DMA,
        ],
    )(_tec_body)
    out = call(src_flat, fz, fy, fx)
    return out.reshape(B, 1, D, H, W)


# speedup vs baseline: 6.3265x; 1.1666x over previous
import functools

import jax
import jax.numpy as jnp
from jax import lax
from jax.experimental import pallas as pl
from jax.experimental.pallas import tpu as pltpu
from jax.experimental.pallas import tpu_sc as plsc

B = 2
D = H = W = 128
V = D * H * W
LANES = 16
NC, NS = 2, 16
NW = NC * NS
DPW = D // (NW // B)

DBLK, HBLK = 4, 16
NVOX = DBLK * HBLK * W
STEPS = NVOX // LANES
HALO = 6
ZWIN = DBLK + 2 * HALO + 2
YWIN = HBLK + 2 * HALO + 2
SLAB = ZWIN * YWIN * W
NSUB = 1024
SSTEPS = NSUB // LANES

f32 = jnp.float32
i32 = jnp.int32


def _floor_frac(pos_i, f):
    pos = pos_i.astype(f32) + f
    t = pos.astype(i32)
    t = t - jnp.where(t.astype(f32) > pos, 1, 0)
    return t, pos - t.astype(f32)


def _tec_body(src_hbm, fz_hbm, fy_hbm, fx_hbm, out_hbm,
              slab, fzv, fyv, fxv, accv,
              zq0, zq1, yq0, yq1, xq0, xq1,
              sz0, sz1, sy0, sy1, sx0, sx1,
              idx2, val2, sem_in, sem_g, sem_out):
    cid = lax.axis_index("c")
    sid = lax.axis_index("s")
    wid = sid * NC + cid
    batch = wid // (NW // B)
    bofs = batch * V
    d_base = (wid % (NW // B)) * DPW
    iota = lax.iota(i32, LANES)

    def axis_corners(pos, lo_clip, win):
        pos = jnp.clip(pos, -1.0, 128.0)
        t = pos.astype(i32)
        t = t - jnp.where(t.astype(f32) > pos, 1, 0)
        t = jnp.minimum(t, D - 1)
        fr = pos - t.astype(f32)
        w0 = jnp.where(t >= 0, 1.0 - fr, 0.0)
        w1 = jnp.where(t <= D - 2, fr, 0.0)
        u = t - lo_clip
        c0 = jnp.clip(u, 0, win - 1)
        c1 = jnp.clip(u + 1, 0, win - 1)
        return c0, c1, w0, w1

    def chunk(c, carry):
        d0 = d_base + (c // (H // HBLK)) * DBLK
        h0 = (c % (H // HBLK)) * HBLK
        z_base = jnp.clip(d0 - HALO, 0, D - ZWIN)
        y_base = jnp.clip(h0 - HALO, 0, H - YWIN)

        @pl.when(c > 0)
        def _drain_out():
            for dd in range(DBLK):
                pltpu.make_async_copy(
                    accv.at[pl.ds(dd * HBLK * W, HBLK * W)],
                    out_hbm.at[pl.ds(bofs + dd * (H * W), HBLK * W)],
                    sem_out).wait()

        copies = []
        for zz in range(ZWIN):
            off = bofs + (z_base + zz) * (H * W) + y_base * W
            copies.append(pltpu.async_copy(
                src_hbm.at[pl.ds(off, YWIN * W)],
                slab.at[pl.ds(zz * YWIN * W, YWIN * W)], sem_in))
        for dd in range(DBLK):
            off = bofs + (d0 + dd) * (H * W) + h0 * W
            t = pl.ds(dd * HBLK * W, HBLK * W)
            copies.append(pltpu.async_copy(
                fz_hbm.at[pl.ds(off, HBLK * W)], fzv.at[t], sem_in))
            copies.append(pltpu.async_copy(
                fy_hbm.at[pl.ds(off, HBLK * W)], fyv.at[t], sem_in))
            copies.append(pltpu.async_copy(
                fx_hbm.at[pl.ds(off, HBLK * W)], fxv.at[t], sem_in))
        for cp in copies:
            cp.wait()

        def fast(i, flag):
            sl = pl.ds(i * LANES, LANES)
            rr = i // (W // LANES)
            hq = h0 + (rr % HBLK)
            dq = d0 + rr // HBLK
            wq = (i % (W // LANES)) * LANES + iota

            fzx = fzv[sl]
            fyx = fyv[sl]
            fxx = fxv[sl]
            half = jnp.float32(HALO)
            ofl = (fzx < -half) | (fzx > half) | (fyx < -half) | (fyx > half)
            flag = flag | jnp.where(ofl, 1, 0)

            zc0, zc1, wz0, wz1 = axis_corners(
                dq.astype(f32) + fzx, z_base, ZWIN)
            yc0, yc1, wy0, wy1 = axis_corners(
                hq.astype(f32) + fyx, y_base, YWIN)
            xp0, xp1, wx0, wx1 = axis_corners(
                wq.astype(f32) + fxx, 0, W)

            zp0 = zc0 * (YWIN * W)
            zp1 = zc1 * (YWIN * W)
            yp0 = yc0 << 7
            yp1 = yc1 << 7

            acc = None
            for zp, wz in ((zp0, wz0), (zp1, wz1)):
                for yp, wy in ((yp0, wy0), (yp1, wy1)):
                    bzy = zp + yp
                    tzy = wz * wy
                    for xp, wx in ((xp0, wx0), (xp1, wx1)):
                        val = plsc.load_gather(slab, [bzy + xp])
                        term = (tzy * wx) * val
                        acc = term if acc is None else acc + term
            accv[sl] = acc
            return flag

        flag = plsc.parallel_loop(
            0, STEPS, 1, unroll=4, carry=jnp.zeros((LANES,), i32))(fast)

        @pl.when(jnp.max(flag) > 0)
        def _slow():
            def sub_body(sub, carry2):
                def pass1(i, carry1):
                    sl = pl.ds(i * LANES, LANES)
                    j = i * LANES + iota
                    wq = j & (W - 1)
                    rr = sub * (NSUB // W) + (j >> 7)
                    hq = h0 + (rr & (HBLK - 1))
                    dq = d0 + (rr >> 4)
                    z0, frz = _floor_frac(dq, fzv[pl.ds(sub * NSUB + i * LANES, LANES)])
                    y0, fry = _floor_frac(hq, fyv[pl.ds(sub * NSUB + i * LANES, LANES)])
                    x0, frx = _floor_frac(wq, fxv[pl.ds(sub * NSUB + i * LANES, LANES)])

                    def parts(c0i, frac, limit, shift, base, r0, r1, w0r, w1r):
                        in0 = (c0i >= 0) & (c0i <= limit)
                        in1 = (c0i >= -1) & (c0i <= limit - 1)
                        r0[sl] = base + (jnp.clip(c0i, 0, limit) << shift)
                        r1[sl] = base + (jnp.clip(c0i + 1, 0, limit) << shift)
                        w0r[sl] = jnp.where(in0, 1.0 - frac, 0.0)
                        w1r[sl] = jnp.where(in1, frac, 0.0)

                    parts(z0, frz, D - 1, 14, bofs, zq0, zq1, sz0, sz1)
                    parts(y0, fry, H - 1, 7, 0, yq0, yq1, sy0, sy1)
                    parts(x0, frx, W - 1, 0, 0, xq0, xq1, sx0, sx1)
                    return carry1

                lax.fori_loop(0, SSTEPS, pass1, 0)

                first = True
                for zq, sz in ((zq0, sz0), (zq1, sz1)):
                    for yq, sy in ((yq0, sy0), (yq1, sy1)):
                        for xq, sx in ((xq0, sx0), (xq1, sx1)):
                            def mkidx(i, carry1, zq=zq, yq=yq, xq=xq):
                                sl = pl.ds(i * LANES, LANES)
                                idx2[sl] = zq[sl] + yq[sl] + xq[sl]
                                return carry1

                            lax.fori_loop(0, SSTEPS, mkidx, 0)
                            pltpu.async_copy(src_hbm.at[idx2], val2, sem_g).wait()

                            def accum(i, carry1, sz=sz, sy=sy, sx=sx, first=first):
                                sl = pl.ds(i * LANES, LANES)
                                osl = pl.ds(sub * NSUB + i * LANES, LANES)
                                contrib = (sz[sl] * sy[sl]) * (sx[sl] * val2[sl])
                                if first:
                                    accv[osl] = contrib
                                else:
                                    accv[osl] = accv[osl] + contrib
                                return carry1

                            lax.fori_loop(0, SSTEPS, accum, 0)
                            first = False
                return carry2

            lax.fori_loop(0, NVOX // NSUB, sub_body, 0)

        for dd in range(DBLK):
            off = bofs + (d0 + dd) * (H * W) + h0 * W
            pltpu.async_copy(
                accv.at[pl.ds(dd * HBLK * W, HBLK * W)],
                out_hbm.at[pl.ds(off, HBLK * W)], sem_out)
        return carry

    lax.fori_loop(0, (DPW // DBLK) * (H // HBLK), chunk, 0)
    for dd in range(DBLK):
        pltpu.make_async_copy(
            accv.at[pl.ds(dd * HBLK * W, HBLK * W)],
            out_hbm.at[pl.ds(bofs + dd * (H * W), HBLK * W)],
            sem_out).wait()


@jax.jit
def kernel(src, flow):
    src_flat = src.reshape(B * V)
    fz = flow[:, 0].reshape(B * V)
    fy = flow[:, 1].reshape(B * V)
    fx = flow[:, 2].reshape(B * V)

    mesh = plsc.VectorSubcoreMesh(core_axis_name="c", subcore_axis_name="s")
    call = functools.partial(
        pl.kernel,
        out_type=jax.ShapeDtypeStruct((B * V,), f32),
        mesh=mesh,
        compiler_params=pltpu.CompilerParams(needs_layout_passes=False),
        scratch_types=[
            pltpu.VMEM((SLAB,), f32),
            pltpu.VMEM((NVOX,), f32),
            pltpu.VMEM((NVOX,), f32),
            pltpu.VMEM((NVOX,), f32),
            pltpu.VMEM((NVOX,), f32),
            pltpu.VMEM((NSUB,), i32),
            pltpu.VMEM((NSUB,), i32),
            pltpu.VMEM((NSUB,), i32),
            pltpu.VMEM((NSUB,), i32),
            pltpu.VMEM((NSUB,), i32),
            pltpu.VMEM((NSUB,), i32),
            pltpu.VMEM((NSUB,), f32),
            pltpu.VMEM((NSUB,), f32),
            pltpu.VMEM((NSUB,), f32),
            pltpu.VMEM((NSUB,), f32),
            pltpu.VMEM((NSUB,), f32),
            pltpu.VMEM((NSUB,), f32),
            pltpu.VMEM((NSUB,), i32),
            pltpu.VMEM((NSUB,), f32),
            pltpu.SemaphoreType.DMA,
            pltpu.SemaphoreType.DMA,
            pltpu.SemaphoreType.DMA,
        ],
    )(_tec_body)
    out = call(src_flat, fz, fy, fx)
    return out.reshape(B, 1, D, H, W)
